# baseline (speedup 1.0000x reference)
import functools

import jax
import jax.numpy as jnp
from jax import lax
from jax.experimental import pallas as pl
from jax.experimental.pallas import tpu as pltpu
from jax.experimental.pallas import tpu_sc as plsc

N = 10000
NPAD = 10240
NC, NS = 2, 16
NW = NC * NS
K = 128
NBUF = 4
EPAD = 327680
JUNK = NPAD - 1
ROWS_PER_TILE = NPAD // NS

NB_H = EPAD // (NS * K)
NB_L = EPAD // (NW * K)
DH = 64

_MESH = plsc.VectorSubcoreMesh(core_axis_name="c", subcore_axis_name="s")
_PARAMS = pltpu.CompilerParams(use_tc_tiling_on_sc=False)


def _fill(ref, nrows, ncols, val):
    def fr(i, carry):
        for jj in range(ncols // 16):
            ref[i, pl.ds(jj * 16, 16)] = jnp.full((16,), val, jnp.float32)
        return carry

    lax.fori_loop(0, nrows, fr, 0)


def _zero_acc(zbuf, acc, s):
    def za(jj, carry):
        pltpu.sync_copy(zbuf, acc.at[pl.ds(s * ROWS_PER_TILE + jj * K, K)])
        return carry

    lax.fori_loop(0, ROWS_PER_TILE // K, za, 0)


def _sweep(z_hbm, idx_src, idx_dst, rows, acc, gsems, ssems, nb):
    ng = nb // NBUF
    for j in range(NBUF):
        pltpu.async_copy(z_hbm.at[idx_src.at[j]], rows[j], gsems[j])

    def group(g, carry):
        scat = []
        for j in range(NBUF):
            b = g * NBUF + j
            pltpu.make_async_copy(z_hbm.at[idx_src.at[b]], rows[j], gsems[j]).wait()
            scat.append(
                pltpu.async_copy(rows[j], acc.at[idx_dst.at[b]], ssems[j], add=True)
            )
        for j in range(NBUF):
            scat[j].wait()

            @pl.when(g + 1 < ng)
            def _():
                bn = (g + 1) * NBUF + j
                pltpu.async_copy(z_hbm.at[idx_src.at[bn]], rows[j], gsems[j])

        return carry

    lax.fori_loop(0, ng, group, 0)


def _make_hop(n_s, n_t):
    n_out = n_s + n_t

    @functools.partial(
        pl.kernel,
        out_type=tuple(
            jax.ShapeDtypeStruct((NC, NPAD, 128), jnp.float32) for _ in range(n_out)
        ),
        mesh=_MESH,
        scratch_types=(
            [
                pltpu.VMEM((NB_L, K), jnp.int32),
                pltpu.VMEM((NB_L, K), jnp.int32),
                pltpu.VMEM((K, 128), jnp.float32),
            ]
            + [pltpu.VMEM_SHARED((NPAD, 128), jnp.float32)]
            + [pltpu.SemaphoreType.DMA]
        ),
        compiler_params=_PARAMS,
    )
    def hop(dstS, srcS, dstT, srcT, tok, *rest):
        del tok
        zs = rest[:n_out]
        outs = rest[n_out:2 * n_out]
        idx_dst, idx_src, rows, acc, gsem = rest[2 * n_out:]
        c = lax.axis_index("c")
        s = lax.axis_index("s")
        w = c * NS + s

        _fill(rows, K, 128, 0.0)

        t = 0
        for dst_hbm, src_hbm, n_dir in ((dstS, srcS, n_s), (dstT, srcT, n_t)):
            pltpu.sync_copy(dst_hbm.at[w], idx_dst)
            pltpu.sync_copy(src_hbm.at[w], idx_src)
            for _ in range(n_dir):
                _zero_acc(rows, acc, s)
                plsc.subcore_barrier()

                def step(b, carry):
                    pltpu.async_copy(zs[t].at[idx_src.at[b]], rows, gsem).wait()
                    pltpu.sync_copy(rows, acc.at[idx_dst.at[b]], add=True)
                    return carry

                lax.fori_loop(0, NB_L, step, 0)
                plsc.subcore_barrier()
                pltpu.sync_copy(
                    acc.at[pl.ds(s * ROWS_PER_TILE, ROWS_PER_TILE)],
                    outs[t].at[c, pl.ds(s * ROWS_PER_TILE, ROWS_PER_TILE)],
                )
                if t + 1 < n_out:
                    _fill(rows, K, 128, 0.0)
                t += 1

    return hop


def _make_lvl():
    @functools.partial(
        pl.kernel,
        out_type=tuple(
            jax.ShapeDtypeStruct((NC, NPAD, 16), jnp.float32) for _ in range(2)
        ),
        mesh=_MESH,
        scratch_types=(
            [
                pltpu.VMEM((NB_L, K), jnp.int32),
                pltpu.VMEM((NB_L, K), jnp.int32),
            ]
            + [pltpu.VMEM((K, 16), jnp.float32) for _ in range(NBUF)]
            + [pltpu.VMEM_SHARED((NPAD, 16), jnp.float32)]
            + [pltpu.SemaphoreType.DMA for _ in range(2 * NBUF)]
        ),
        compiler_params=_PARAMS,
    )
    def lvl(dstS, srcS, dstT, srcT, tok, zS, zT, outS, outT, *rest):
        del tok
        idx_dst, idx_src = rest[:2]
        rows = rest[2:2 + NBUF]
        acc = rest[2 + NBUF]
        gsems = rest[3 + NBUF:3 + 2 * NBUF]
        ssems = rest[3 + 2 * NBUF:]
        c = lax.axis_index("c")
        s = lax.axis_index("s")
        w = c * NS + s

        _fill(rows[0], K, 16, 0.0)
        first = True
        for dst_hbm, src_hbm, z, out in ((dstS, srcS, zS, outS),
                                         (dstT, srcT, zT, outT)):
            pltpu.sync_copy(dst_hbm.at[w], idx_dst)
            pltpu.sync_copy(src_hbm.at[w], idx_src)
            if not first:
                _fill(rows[0], K, 16, 0.0)
            _zero_acc(rows[0], acc, s)
            plsc.subcore_barrier()
            _sweep(z, idx_src, idx_dst, rows, acc, gsems, ssems, NB_L)
            plsc.subcore_barrier()
            pltpu.sync_copy(
                acc.at[pl.ds(s * ROWS_PER_TILE, ROWS_PER_TILE)],
                out.at[c, pl.ds(s * ROWS_PER_TILE, ROWS_PER_TILE)],
            )
            first = False

    return lvl


_lvl16 = _make_lvl()
_hop1 = _make_hop(1, 1)


def _tc_combine(hcat, wcat, bias):
    BN = 512

    def body(h_ref, w_ref, b_ref, o_ref):
        o_ref[...] = (
            jnp.dot(h_ref[...], w_ref[...], preferred_element_type=jnp.float32)
            + b_ref[...]
        )

    return pl.pallas_call(
        body,
        grid=(NPAD // BN,),
        in_specs=[
            pl.BlockSpec((BN, 768), lambda i: (i, 0)),
            pl.BlockSpec((768, 128), lambda i: (0, 0)),
            pl.BlockSpec((1, 128), lambda i: (0, 0)),
        ],
        out_specs=pl.BlockSpec((BN, 128), lambda i: (i, 0)),
        out_shape=jax.ShapeDtypeStruct((NPAD, 128), jnp.float32),
    )(hcat, wcat, bias)


def _inv_sqrt(d):
    return jnp.where(d > 0, 1.0 / jnp.sqrt(jnp.where(d > 0, d, 1.0)), 0.0)


def _col16(*cols):
    z = [c[:, None] for c in cols]
    z.append(jnp.zeros((N, 16 - len(cols)), jnp.float32))
    return jnp.concatenate(z, axis=1)


def kernel(x, edge_index, W_sd, b_sd, W_ds, b_ds, W0, b0, W1, b1, W2, b2,
           W3, b3, alpha, beta, gama):
    row, col = edge_index[0], edge_index[1]
    pad = EPAD - row.shape[0]
    junk = jnp.full((pad,), JUNK, jnp.int32)
    zero = jnp.zeros((pad,), jnp.int32)
    rowp = jnp.concatenate([row, junk])
    colp_d = jnp.concatenate([col, junk])
    colp_s = jnp.concatenate([col, zero])
    rowp_s = jnp.concatenate([row, zero])
    idxL = (rowp.reshape(NW, NB_L, K), colp_s.reshape(NW, NB_L, K),
            colp_d.reshape(NW, NB_L, K), rowp_s.reshape(NW, NB_L, K))

    def both(o):
        return (o[0] + o[1])[:N]

    zt = jnp.zeros((8,), jnp.float32)

    ones16 = jnp.ones((N, 16), jnp.float32)
    og, ig = _lvl16(*idxL, zt, ones16, ones16)
    out_deg = both(og)[:, 0]
    in_deg = both(ig)[:, 0]
    dout = _inv_sqrt(out_deg)
    din = _inv_sqrt(in_deg)

    qo, po = _lvl16(*idxL, zt, _col16(din), _col16(dout))
    q = dout * both(qo)[:, 0]
    p = din * both(po)[:, 0]

    r13o, r24o = _lvl16(*idxL, zt, _col16(din * p, din * q),
                        _col16(dout * q, dout * p))
    r13 = both(r13o)
    r24 = both(r24o)
    r1 = dout * r13[:, 0]
    r3 = dout * r13[:, 1]
    r2 = din * r24[:, 0]
    r4 = din * r24[:, 1]
    c1, c2, c3, c4 = _inv_sqrt(r1), _inv_sqrt(r2), _inv_sqrt(r3), _inv_sqrt(r4)

    u1o, u2o = _hop1(*idxL, zt,
                     din[:, None] * x, dout[:, None] * x)
    v2o, v1o = _hop1(*idxL, zt,
                     (din * c2)[:, None] * x, (dout * c1)[:, None] * x)
    v3o, v4o = _hop1(*idxL, zt,
                     (din * c4)[:, None] * x, (dout * c3)[:, None] * x)
    U1, V2, V3 = both(u1o), both(v2o), both(v3o)
    U2, V1, V4 = both(u2o), both(v1o), both(v4o)

    h3o, h4o = _hop1(*idxL, zt,
                     (din * din)[:, None] * V1, (dout * dout)[:, None] * V2)
    h5o, h6o = _hop1(*idxL, zt,
                     (din * dout)[:, None] * V3, (dout * din)[:, None] * V4)
    H3c, H5c, H4c, H6c = both(h3o), both(h5o), both(h4o), both(h6o)

    H1 = dout[:, None] * U1
    H2 = din[:, None] * U2
    H3 = (c1 * dout)[:, None] * H3c
    H4 = (c2 * din)[:, None] * H4c
    H5 = (c3 * dout)[:, None] * H5c
    H6 = (c4 * din)[:, None] * H6c

    hcat = jnp.concatenate([H1, H2, H3, H4, H5, H6], axis=1)
    hcat = jnp.pad(hcat, ((0, NPAD - N), (0, 0)))
    a, b, g = alpha, beta, gama
    wcat = jnp.concatenate([
        a * W_sd.T, (1.0 - a) * W_ds.T,
        b * W0.T, (1.0 - b) * W1.T,
        g * W2.T, (1.0 - g) * W3.T,
    ], axis=0)
    bias = (a * b_sd + (1.0 - a) * b_ds + b * b0 + (1.0 - b) * b1
            + g * b2 + (1.0 - g) * b3)[None, :]

    return _tc_combine(hcat, wcat, bias)[:N]


# speedup vs baseline: 1.3223x; 1.3223x over previous
import functools

import jax
import jax.numpy as jnp
from jax import lax
from jax.experimental import pallas as pl
from jax.experimental.pallas import tpu as pltpu
from jax.experimental.pallas import tpu_sc as plsc

N = 10000
NPAD = 10240
NC, NS = 2, 16
NW = NC * NS
K = 64
NB = 160
NPAIR = NB // 2
EPAD = 327680
JUNK = NPAD - 1
RPT = NPAD // NS

_MESH = plsc.VectorSubcoreMesh(core_axis_name="c", subcore_axis_name="s")
_PARAMS = pltpu.CompilerParams(use_tc_tiling_on_sc=False,
                               needs_layout_passes=False)


def _fill_zero(ref, nrows, ncols):
    def fr(i, carry):
        for jj in range(ncols // 16):
            ref[i, pl.ds(jj * 16, 16)] = jnp.zeros((16,), jnp.float32)
        return carry

    lax.fori_loop(0, nrows, fr, 0)


def _zero_acc(zbuf, acc, s):
    def za(jj, carry):
        pltpu.sync_copy(zbuf, acc.at[pl.ds(s * RPT + jj * K, K)])
        return carry

    lax.fori_loop(0, RPT // K, za, 0)


def _make_hop():
    @functools.partial(
        pl.kernel,
        out_type=jax.ShapeDtypeStruct((NC, NPAD, 128), jnp.float32),
        mesh=_MESH,
        scratch_types=[
            pltpu.VMEM((NB, K), jnp.int32),
            pltpu.VMEM((NB, K), jnp.int32),
            pltpu.VMEM((K, 64), jnp.int32),
            pltpu.VMEM((K, 64), jnp.int32),
            pltpu.VMEM((K, 128), jnp.float32),
            pltpu.VMEM_SHARED((NPAD, 128), jnp.float32),
            pltpu.SemaphoreType.DMA,
            pltpu.SemaphoreType.DMA,
        ],
        compiler_params=_PARAMS,
    )
    def hop(dst_hbm, src_hbm, zp, out, idx_dst, idx_src, p0, p1, rows, acc, g0, g1):
        c = lax.axis_index("c")
        s = lax.axis_index("s")
        w = c * NS + s

        pltpu.sync_copy(dst_hbm.at[w], idx_dst)
        pltpu.sync_copy(src_hbm.at[w], idx_src)
        _fill_zero(rows, K, 128)
        _zero_acc(rows, acc, s)
        pltpu.async_copy(zp.at[idx_src.at[0]], p0, g0)
        plsc.subcore_barrier()

        mask = jnp.full((16,), -65536, jnp.int32)

        def convert(src_p):
            def ce(e, carry):
                for jj in range(4):
                    v = src_p[e, pl.ds(jj * 16, 16)]
                    rows[e, pl.ds(jj * 32, 16)] = plsc.bitcast(
                        lax.shift_left(v, 16), jnp.float32)
                    rows[e, pl.ds(jj * 32 + 16, 16)] = plsc.bitcast(
                        v & mask, jnp.float32)
                return carry

            lax.fori_loop(0, K, ce, 0)

        def pair(i, carry):
            b0 = 2 * i
            b1 = 2 * i + 1
            pltpu.make_async_copy(zp.at[idx_src.at[b0]], p0, g0).wait()
            pltpu.async_copy(zp.at[idx_src.at[b1]], p1, g1)
            convert(p0)
            pltpu.sync_copy(rows, acc.at[idx_dst.at[b0]], add=True)
            pltpu.make_async_copy(zp.at[idx_src.at[b1]], p1, g1).wait()

            @pl.when(i + 1 < NPAIR)
            def _():
                pltpu.async_copy(zp.at[idx_src.at[b0 + 2]], p0, g0)

            convert(p1)
            pltpu.sync_copy(rows, acc.at[idx_dst.at[b1]], add=True)
            return carry

        lax.fori_loop(0, NPAIR, pair, 0)
        plsc.subcore_barrier()
        pltpu.sync_copy(
            acc.at[pl.ds(s * RPT, RPT)],
            out.at[c, pl.ds(s * RPT, RPT)],
        )

    return hop


def _make_lvl():
    @functools.partial(
        pl.kernel,
        out_type=jax.ShapeDtypeStruct((NC, NPAD, 16), jnp.float32),
        mesh=_MESH,
        scratch_types=[
            pltpu.VMEM((NB, K), jnp.int32),
            pltpu.VMEM((NB, K), jnp.int32),
            pltpu.VMEM((K, 16), jnp.float32),
            pltpu.VMEM_SHARED((NPAD, 16), jnp.float32),
            pltpu.SemaphoreType.DMA,
        ],
        compiler_params=_PARAMS,
    )
    def lvl(dst_hbm, src_hbm, z, out, idx_dst, idx_src, rows, acc, gsem):
        c = lax.axis_index("c")
        s = lax.axis_index("s")
        w = c * NS + s

        pltpu.sync_copy(dst_hbm.at[w], idx_dst)
        pltpu.sync_copy(src_hbm.at[w], idx_src)
        _fill_zero(rows, K, 16)
        _zero_acc(rows, acc, s)
        plsc.subcore_barrier()

        def step(b, carry):
            pltpu.async_copy(z.at[idx_src.at[b]], rows, gsem).wait()
            pltpu.sync_copy(rows, acc.at[idx_dst.at[b]], add=True)
            return carry

        lax.fori_loop(0, NB, step, 0)
        plsc.subcore_barrier()
        pltpu.sync_copy(
            acc.at[pl.ds(s * RPT, RPT)],
            out.at[c, pl.ds(s * RPT, RPT)],
        )

    return lvl


_hop = _make_hop()
_lvl = _make_lvl()


def _pack(z):
    zb = z.astype(jnp.bfloat16).reshape(N, 4, 2, 16)
    zpair = jnp.stack([zb[:, :, 0, :], zb[:, :, 1, :]], axis=-1)
    return lax.bitcast_convert_type(zpair, jnp.int32).reshape(N, 64)


def _tc_combine(hcat, wcat, bias):
    BN = 512

    def body(h_ref, w_ref, b_ref, o_ref):
        o_ref[...] = (
            jnp.dot(h_ref[...], w_ref[...], preferred_element_type=jnp.float32)
            + b_ref[...]
        )

    return pl.pallas_call(
        body,
        grid=(NPAD // BN,),
        in_specs=[
            pl.BlockSpec((BN, 768), lambda i: (i, 0)),
            pl.BlockSpec((768, 128), lambda i: (0, 0)),
            pl.BlockSpec((1, 128), lambda i: (0, 0)),
        ],
        out_specs=pl.BlockSpec((BN, 128), lambda i: (i, 0)),
        out_shape=jax.ShapeDtypeStruct((NPAD, 128), jnp.float32),
    )(hcat, wcat, bias)


def _inv_sqrt(d):
    return jnp.where(d > 0, 1.0 / jnp.sqrt(jnp.where(d > 0, d, 1.0)), 0.0)


def _col16(*cols):
    z = [c[:, None] for c in cols]
    z.append(jnp.zeros((N, 16 - len(cols)), jnp.float32))
    return jnp.concatenate(z, axis=1)


def kernel(x, edge_index, W_sd, b_sd, W_ds, b_ds, W0, b0, W1, b1, W2, b2,
           W3, b3, alpha, beta, gama):
    row, col = edge_index[0], edge_index[1]
    pad = EPAD - row.shape[0]
    junk = jnp.full((pad,), JUNK, jnp.int32)
    zero = jnp.zeros((pad,), jnp.int32)
    dstS = jnp.concatenate([row, junk]).reshape(NW, NB, K)
    srcS = jnp.concatenate([col, zero]).reshape(NW, NB, K)
    dstT = jnp.concatenate([col, junk]).reshape(NW, NB, K)
    srcT = jnp.concatenate([row, zero]).reshape(NW, NB, K)

    def S16(z):
        p = _lvl(dstS, srcS, z)
        return (p[0] + p[1])[:N]

    def T16(z):
        p = _lvl(dstT, srcT, z)
        return (p[0] + p[1])[:N]

    def S128(z):
        p = _hop(dstS, srcS, _pack(z))
        return (p[0] + p[1])[:N]

    def T128(z):
        p = _hop(dstT, srcT, _pack(z))
        return (p[0] + p[1])[:N]

    ones16 = jnp.ones((N, 16), jnp.float32)
    out_deg = S16(ones16)[:, 0]
    in_deg = T16(ones16)[:, 0]
    dout = _inv_sqrt(out_deg)
    din = _inv_sqrt(in_deg)

    q = dout * S16(_col16(din))[:, 0]
    p = din * T16(_col16(dout))[:, 0]

    r13 = S16(_col16(din * p, din * q))
    r1 = dout * r13[:, 0]
    r3 = dout * r13[:, 1]
    r24 = T16(_col16(dout * q, dout * p))
    r2 = din * r24[:, 0]
    r4 = din * r24[:, 1]
    c1, c2, c3, c4 = _inv_sqrt(r1), _inv_sqrt(r2), _inv_sqrt(r3), _inv_sqrt(r4)

    U1 = S128(din[:, None] * x)
    V2 = S128((din * c2)[:, None] * x)
    V3 = S128((din * c4)[:, None] * x)
    U2 = T128(dout[:, None] * x)
    V1 = T128((dout * c1)[:, None] * x)
    V4 = T128((dout * c3)[:, None] * x)

    H3c = S128((din * din)[:, None] * V1)
    H5c = S128((din * dout)[:, None] * V3)
    H4c = T128((dout * dout)[:, None] * V2)
    H6c = T128((dout * din)[:, None] * V4)

    H1 = dout[:, None] * U1
    H2 = din[:, None] * U2
    H3 = (c1 * dout)[:, None] * H3c
    H4 = (c2 * din)[:, None] * H4c
    H5 = (c3 * dout)[:, None] * H5c
    H6 = (c4 * din)[:, None] * H6c

    hcat = jnp.concatenate([H1, H2, H3, H4, H5, H6], axis=1)
    hcat = jnp.pad(hcat, ((0, NPAD - N), (0, 0)))
    a, b, g = alpha, beta, gama
    wcat = jnp.concatenate([
        a * W_sd.T, (1.0 - a) * W_ds.T,
        b * W0.T, (1.0 - b) * W1.T,
        g * W2.T, (1.0 - g) * W3.T,
    ], axis=0)
    bias = (a * b_sd + (1.0 - a) * b_ds + b * b0 + (1.0 - b) * b1
            + g * b2 + (1.0 - g) * b3)[None, :]

    return _tc_combine(hcat, wcat, bias)[:N]


# speedup vs baseline: 1.4504x; 1.0969x over previous
import functools

import jax
import jax.numpy as jnp
from jax import lax
from jax.experimental import pallas as pl
from jax.experimental.pallas import tpu as pltpu
from jax.experimental.pallas import tpu_sc as plsc

N = 10000
NPAD = 10240
NC, NS = 2, 16
NW = NC * NS
K = 128
NB = 79
EPT = NB * K
EPAD = NW * EPT
JUNK = NPAD - 1
ROWS_PER_TILE = NPAD // NS


def _make_spmm(D):
    mesh = plsc.VectorSubcoreMesh(core_axis_name="c", subcore_axis_name="s")

    @functools.partial(
        pl.kernel,
        out_type=jax.ShapeDtypeStruct((NC, NPAD, D), jnp.float32),
        mesh=mesh,
        scratch_types=[
            pltpu.VMEM((NB, K), jnp.int32),
            pltpu.VMEM((NB, K), jnp.int32),
            pltpu.VMEM((K, D), jnp.float32),
            pltpu.VMEM_SHARED((NPAD, D), jnp.float32),
            pltpu.SemaphoreType.DMA,
        ],
        compiler_params=pltpu.CompilerParams(use_tc_tiling_on_sc=False),
    )
    def spmm(dst_hbm, src_hbm, z_hbm, out_hbm, idx_dst, idx_src, rows, acc, gsem):
        c = lax.axis_index("c")
        s = lax.axis_index("s")
        w = c * NS + s

        pltpu.sync_copy(dst_hbm.at[w], idx_dst)
        pltpu.sync_copy(src_hbm.at[w], idx_src)

        def zrow(i, carry):
            for j in range(D // 16):
                rows[i, pl.ds(j * 16, 16)] = jnp.zeros((16,), jnp.float32)
            return carry

        lax.fori_loop(0, K, zrow, 0)

        def zacc(j, carry):
            pltpu.sync_copy(rows, acc.at[pl.ds(s * ROWS_PER_TILE + j * K, K)])
            return carry

        lax.fori_loop(0, ROWS_PER_TILE // K, zacc, 0)
        plsc.subcore_barrier()

        def step(b, carry):
            pltpu.async_copy(z_hbm.at[idx_src.at[b]], rows, gsem).wait()
            pltpu.sync_copy(rows, acc.at[idx_dst.at[b]], add=True)
            return carry

        lax.fori_loop(0, NB, step, 0)
        plsc.subcore_barrier()

        pltpu.sync_copy(
            acc.at[pl.ds(s * ROWS_PER_TILE, ROWS_PER_TILE)],
            out_hbm.at[c, pl.ds(s * ROWS_PER_TILE, ROWS_PER_TILE)],
        )

    return spmm


_spmm16 = _make_spmm(16)
_spmm128 = _make_spmm(128)


def _tc_combine(hcat, wcat, bias):
    BN = 512

    def body(h_ref, w_ref, b_ref, o_ref):
        o_ref[...] = (
            jnp.dot(h_ref[...], w_ref[...], preferred_element_type=jnp.float32)
            + b_ref[...]
        )

    return pl.pallas_call(
        body,
        grid=(NPAD // BN,),
        in_specs=[
            pl.BlockSpec((BN, 768), lambda i: (i, 0)),
            pl.BlockSpec((768, 128), lambda i: (0, 0)),
            pl.BlockSpec((1, 128), lambda i: (0, 0)),
        ],
        out_specs=pl.BlockSpec((BN, 128), lambda i: (i, 0)),
        out_shape=jax.ShapeDtypeStruct((NPAD, 128), jnp.float32),
    )(hcat, wcat, bias)


def _inv_sqrt(d):
    return jnp.where(d > 0, 1.0 / jnp.sqrt(jnp.where(d > 0, d, 1.0)), 0.0)


def _col16(*cols):
    z = [c[:, None] for c in cols]
    z.append(jnp.zeros((N, 16 - len(cols)), jnp.float32))
    return jnp.concatenate(z, axis=1)


def kernel(x, edge_index, W_sd, b_sd, W_ds, b_ds, W0, b0, W1, b1, W2, b2,
           W3, b3, alpha, beta, gama):
    row, col = edge_index[0], edge_index[1]
    pad = EPAD - row.shape[0]
    junk = jnp.full((pad,), JUNK, jnp.int32)
    zero = jnp.zeros((pad,), jnp.int32)
    dstS = jnp.concatenate([row, junk]).reshape(NW, NB, K)
    srcS = jnp.concatenate([col, zero]).reshape(NW, NB, K)
    dstT = jnp.concatenate([col, junk]).reshape(NW, NB, K)
    srcT = jnp.concatenate([row, zero]).reshape(NW, NB, K)

    def S16(z):
        p = _spmm16(dstS, srcS, z)
        return (p[0] + p[1])[:N]

    def T16(z):
        p = _spmm16(dstT, srcT, z)
        return (p[0] + p[1])[:N]

    def S128(z):
        p = _spmm128(dstS, srcS, z)
        return (p[0] + p[1])[:N]

    def T128(z):
        p = _spmm128(dstT, srcT, z)
        return (p[0] + p[1])[:N]

    ones16 = jnp.ones((N, 16), jnp.float32)
    out_deg = S16(ones16)[:, 0]
    in_deg = T16(ones16)[:, 0]
    dout = _inv_sqrt(out_deg)
    din = _inv_sqrt(in_deg)

    q = dout * S16(_col16(din))[:, 0]
    p = din * T16(_col16(dout))[:, 0]

    r13 = S16(_col16(din * p, din * q))
    r1 = dout * r13[:, 0]
    r3 = dout * r13[:, 1]
    r24 = T16(_col16(dout * q, dout * p))
    r2 = din * r24[:, 0]
    r4 = din * r24[:, 1]
    c1, c2, c3, c4 = _inv_sqrt(r1), _inv_sqrt(r2), _inv_sqrt(r3), _inv_sqrt(r4)

    U1 = S128(din[:, None] * x)
    V2 = S128((din * c2)[:, None] * x)
    V3 = S128((din * c4)[:, None] * x)
    U2 = T128(dout[:, None] * x)
    V1 = T128((dout * c1)[:, None] * x)
    V4 = T128((dout * c3)[:, None] * x)

    H3c = S128((din * din)[:, None] * V1)
    H5c = S128((din * dout)[:, None] * V3)
    H4c = T128((dout * dout)[:, None] * V2)
    H6c = T128((dout * din)[:, None] * V4)

    H1 = dout[:, None] * U1
    H2 = din[:, None] * U2
    H3 = (c1 * dout)[:, None] * H3c
    H4 = (c2 * din)[:, None] * H4c
    H5 = (c3 * dout)[:, None] * H5c
    H6 = (c4 * din)[:, None] * H6c

    hcat = jnp.concatenate([H1, H2, H3, H4, H5, H6], axis=1)
    hcat = jnp.pad(hcat, ((0, NPAD - N), (0, 0)))
    a, b, g = alpha, beta, gama
    wcat = jnp.concatenate([
        a * W_sd.T, (1.0 - a) * W_ds.T,
        b * W0.T, (1.0 - b) * W1.T,
        g * W2.T, (1.0 - g) * W3.T,
    ], axis=0)
    bias = (a * b_sd + (1.0 - a) * b_ds + b * b0 + (1.0 - b) * b1
            + g * b2 + (1.0 - g) * b3)[None, :]

    return _tc_combine(hcat, wcat, bias)[:N]


# speedup vs baseline: 1.4516x; 1.0008x over previous
import functools

import jax
import jax.numpy as jnp
from jax import lax
from jax.experimental import pallas as pl
from jax.experimental.pallas import tpu as pltpu
from jax.experimental.pallas import tpu_sc as plsc

N = 10000
NPAD = 10240
NC, NS = 2, 16
NW = NC * NS
K = 128
NB = 79
EPT = NB * K
EPAD = NW * EPT
JUNK = NPAD - 1
ROWS_PER_TILE = NPAD // NS


def _make_spmm(D):
    mesh = plsc.VectorSubcoreMesh(core_axis_name="c", subcore_axis_name="s")

    @functools.partial(
        pl.kernel,
        out_type=jax.ShapeDtypeStruct((NC, NPAD, D), jnp.float32),
        mesh=mesh,
        scratch_types=[
            pltpu.VMEM((NB, K), jnp.int32),
            pltpu.VMEM((NB, K), jnp.int32),
            pltpu.VMEM((K, D), jnp.float32),
            pltpu.VMEM_SHARED((NPAD, D), jnp.float32),
            pltpu.SemaphoreType.DMA,
        ],
        compiler_params=pltpu.CompilerParams(use_tc_tiling_on_sc=False),
    )
    def spmm(dst_hbm, src_hbm, z_hbm, out_hbm, idx_dst, idx_src, rows, acc, gsem):
        c = lax.axis_index("c")
        s = lax.axis_index("s")
        w = c * NS + s

        pltpu.sync_copy(dst_hbm.at[w], idx_dst)
        pltpu.sync_copy(src_hbm.at[w], idx_src)

        def zrow(i, carry):
            for j in range(D // 16):
                rows[i, pl.ds(j * 16, 16)] = jnp.zeros((16,), jnp.float32)
            return carry

        lax.fori_loop(0, K, zrow, 0)

        def zacc(j, carry):
            pltpu.sync_copy(rows, acc.at[pl.ds(s * ROWS_PER_TILE + j * K, K)])
            return carry

        lax.fori_loop(0, ROWS_PER_TILE // K, zacc, 0)
        plsc.subcore_barrier()

        def step(b, carry):
            pltpu.async_copy(z_hbm.at[idx_src.at[b]], rows, gsem).wait()
            pltpu.sync_copy(rows, acc.at[idx_dst.at[b]], add=True)
            return carry

        lax.fori_loop(0, NB, step, 0)
        plsc.subcore_barrier()

        pltpu.sync_copy(
            acc.at[pl.ds(s * ROWS_PER_TILE, ROWS_PER_TILE)],
            out_hbm.at[c, pl.ds(s * ROWS_PER_TILE, ROWS_PER_TILE)],
        )

    return spmm


_spmm16 = _make_spmm(16)
_spmm128 = _make_spmm(128)


def _tc_combine(hcat, wcat, bias):
    BN = 512

    def body(h_ref, w_ref, b_ref, o_ref):
        o_ref[...] = (
            jnp.dot(h_ref[...], w_ref[...], preferred_element_type=jnp.float32)
            + b_ref[...]
        )

    return pl.pallas_call(
        body,
        grid=(NPAD // BN,),
        in_specs=[
            pl.BlockSpec((BN, 768), lambda i: (i, 0)),
            pl.BlockSpec((768, 128), lambda i: (0, 0)),
            pl.BlockSpec((1, 128), lambda i: (0, 0)),
        ],
        out_specs=pl.BlockSpec((BN, 128), lambda i: (i, 0)),
        out_shape=jax.ShapeDtypeStruct((NPAD, 128), jnp.float32),
    )(hcat, wcat, bias)


def _inv_sqrt(d):
    return jnp.where(d > 0, 1.0 / jnp.sqrt(jnp.where(d > 0, d, 1.0)), 0.0)


def _col16(*cols):
    z = [c[:, None] for c in cols]
    z.append(jnp.zeros((N, 16 - len(cols)), jnp.float32))
    return jnp.concatenate(z, axis=1)


def kernel(x, edge_index, W_sd, b_sd, W_ds, b_ds, W0, b0, W1, b1, W2, b2,
           W3, b3, alpha, beta, gama):
    row, col = edge_index[0], edge_index[1]
    pad = EPAD - row.shape[0]
    junk = jnp.full((pad,), JUNK, jnp.int32)
    zero = jnp.zeros((pad,), jnp.int32)
    dstS = jnp.concatenate([row, junk]).reshape(NW, NB, K)
    srcS = jnp.concatenate([col, zero]).reshape(NW, NB, K)
    dstT = jnp.concatenate([col, junk]).reshape(NW, NB, K)
    srcT = jnp.concatenate([row, zero]).reshape(NW, NB, K)

    def S16(z):
        p = _spmm16(dstS, srcS, z)
        return (p[0] + p[1])[:N]

    def T16(z):
        p = _spmm16(dstT, srcT, z)
        return (p[0] + p[1])[:N]

    def S128(z):
        p = _spmm128(dstS, srcS, z)
        return (p[0] + p[1])[:N]

    def T128(z):
        p = _spmm128(dstT, srcT, z)
        return (p[0] + p[1])[:N]

    ones16 = jnp.ones((N, 16), jnp.float32)
    out_deg = S16(ones16)[:, 0]
    in_deg = T16(ones16)[:, 0]
    dout = _inv_sqrt(out_deg)
    din = _inv_sqrt(in_deg)

    U1 = S128(din[:, None] * x)
    U2 = T128(dout[:, None] * x)

    q = dout * S16(_col16(din))[:, 0]
    p = din * T16(_col16(dout))[:, 0]

    r13 = S16(_col16(din * p, din * q))
    r1 = dout * r13[:, 0]
    r3 = dout * r13[:, 1]
    r24 = T16(_col16(dout * q, dout * p))
    r2 = din * r24[:, 0]
    r4 = din * r24[:, 1]
    c1, c2, c3, c4 = _inv_sqrt(r1), _inv_sqrt(r2), _inv_sqrt(r3), _inv_sqrt(r4)

    V2 = S128((din * c2)[:, None] * x)
    V3 = S128((din * c4)[:, None] * x)
    V1 = T128((dout * c1)[:, None] * x)
    V4 = T128((dout * c3)[:, None] * x)

    H3c = S128((din * din)[:, None] * V1)
    H5c = S128((din * dout)[:, None] * V3)
    H4c = T128((dout * dout)[:, None] * V2)
    H6c = T128((dout * din)[:, None] * V4)

    H1 = dout[:, None] * U1
    H2 = din[:, None] * U2
    H3 = (c1 * dout)[:, None] * H3c
    H4 = (c2 * din)[:, None] * H4c
    H5 = (c3 * dout)[:, None] * H5c
    H6 = (c4 * din)[:, None] * H6c

    hcat = jnp.concatenate([H1, H2, H3, H4, H5, H6], axis=1)
    hcat = jnp.pad(hcat, ((0, NPAD - N), (0, 0)))
    a, b, g = alpha, beta, gama
    wcat = jnp.concatenate([
        a * W_sd.T, (1.0 - a) * W_ds.T,
        b * W0.T, (1.0 - b) * W1.T,
        g * W2.T, (1.0 - g) * W3.T,
    ], axis=0)
    bias = (a * b_sd + (1.0 - a) * b_ds + b * b0 + (1.0 - b) * b1
            + g * b2 + (1.0 - g) * b3)[None, :]

    return _tc_combine(hcat, wcat, bias)[:N]
